# Initial kernel scaffold; baseline (speedup 1.0000x reference)
import jax
import jax.numpy as jnp
from jax.experimental import pallas as pl


def kernel(Vx, l, dummy, l2, l3, l4, laspp, wv, bv, bng, bnb, bnm, bnvv, wt_, bt, ew, eb, ebng, ebnb, ebnm, ebnv, lw, lb, igw, igb, igg, igbb, igm, igvv, qkvw, qkvb, pw, pb):
    raise NotImplementedError("write your pallas kernel here")


# speedup vs baseline: 6.3030x; 6.3030x over previous
import jax
import jax.numpy as jnp
from jax.experimental import pallas as pl
from jax.experimental.pallas import tpu as pltpu

_B, _H, _W = 4, 32, 32
_N = _H * _W
_T = 77
_TP = 128
_DIM = 256
_NH = 4
_CH = 64
_K = 9


def _dot(a, b):
    return jax.lax.dot_general(a.astype(jnp.bfloat16), b.astype(jnp.bfloat16),
                               (((1,), (0,)), ((), ())),
                               preferred_element_type=jnp.float32)


def _dotp(a, b, prec):
    return jax.lax.dot_general(a, b, (((1,), (0,)), ((), ())),
                               preferred_element_type=jnp.float32,
                               precision=prec)


def _dot_hi(a, b):
    return jax.lax.dot_general(a, b, (((1,), (0,)), ((), ())),
                               preferred_element_type=jnp.float32,
                               precision=jax.lax.Precision.HIGHEST)


def _convv_body(xp_ref, w_ref, sb_ref, out_ref):
    taps = []
    for dy in range(3):
        for dx in range(3):
            x = xp_ref[0, dy:dy + _H, dx * 1024:(dx + _W) * 1024]
            taps.append(x.reshape(_N, 1024).astype(jnp.bfloat16))
    xcat = jnp.concatenate(taps, axis=1)
    wb = w_ref[...].astype(jnp.bfloat16)
    acc = None
    for c in range(0, 9216, 512):
        p = jax.lax.dot_general(xcat[:, c:c + 512], wb[c:c + 512],
                                (((1,), (0,)), ((), ())),
                                preferred_element_type=jnp.float32)
        acc = p if acc is None else acc + p
    t = acc + sb_ref[0, :][None, :]
    t = (t - sb_ref[1, :][None, :]) / jnp.sqrt(sb_ref[2, :][None, :] + 1e-5) \
        * sb_ref[3, :][None, :] + sb_ref[4, :][None, :]
    out_ref[0] = jnp.maximum(t, 0.0)


def _convt_body(lt_ref, w_ref, bt_ref, l2_ref, l3_ref, l4_ref, la_ref,
                out_ref):
    lc = _dot(w_ref[...], lt_ref[0]) + bt_ref[...]
    cmask = jax.lax.broadcasted_iota(jnp.int32, (1, _TP), 1) < _T
    lc = jnp.where(cmask, lc, 0.0)
    mu = jnp.sum(lc, axis=1, keepdims=True) / float(_T)
    dd = jnp.where(cmask, lc - mu, 0.0)
    var = jnp.sum(dd * dd, axis=1, keepdims=True) / float(_T)
    y = jnp.maximum((lc - mu) / jnp.sqrt(var + 1e-5), 0.0)
    y = jnp.where(cmask, y, 0.0)
    out_ref[0] = y + l2_ref[0] + l3_ref[0] + l4_ref[0] + la_ref[0]


def _knn_body(v_ref, l_ref, ewt_ref, prm_ref, x_ref, f_ref, featp_ref):
    xh = v_ref[0, 0]
    yh = l_ref[0, 0]
    xn = xh / jnp.clip(jnp.sqrt(jnp.sum(xh * xh, axis=1, keepdims=True)),
                       1e-12, None)
    yn = yh / jnp.clip(jnp.sqrt(jnp.sum(yh * yh, axis=1, keepdims=True)),
                       1e-12, None)
    xsq = jnp.sum(xn * xn, axis=1, keepdims=True)
    ysq = jnp.sum(yn * yn, axis=1)
    dist = xsq - 2.0 * _dot(xn, yn.T) + ysq[None, :]
    colid = jax.lax.broadcasted_iota(jnp.int32, (_N, _TP), 1)
    neg = jnp.where(colid < _T, -dist, -jnp.inf)
    slots = []
    for _ in range(_K):
        m = jnp.max(neg, axis=1, keepdims=True)
        cand = jnp.where(neg == m, colid, _TP)
        idx = jnp.min(cand, axis=1, keepdims=True)
        sel = colid == idx
        slots.append(_dot_hi(sel.astype(jnp.float32), yh))
        neg = jnp.where(sel, -jnp.inf, neg)
    blocks = []
    for j in range(_K):
        blocks.append(xh)
        blocks.append(slots[j] - xh)
    featc = jnp.concatenate(blocks, axis=1)
    featp_ref[...] = jnp.zeros((1032, (_K + 2) * 128), jnp.float32)
    featp_ref[1:_N + 1, 128:(_K + 1) * 128] = featc
    hm = None
    for kk in range(_K):
        xcat = jnp.concatenate(
            [featp_ref[dn:dn + _N, kk * 128:(kk + 3) * 128]
             for dn in range(3)], axis=1)
        acc = _dot(xcat, ewt_ref[0])
        t = acc + prm_ref[0, 0, :][None, :]
        t = (t - prm_ref[0, 1, :][None, :]) \
            / jnp.sqrt(prm_ref[0, 2, :][None, :] + 1e-5) \
            * prm_ref[0, 3, :][None, :] + prm_ref[0, 4, :][None, :]
        hj = jnp.maximum(t, 0.0)
        hm = hj if hm is None else jnp.maximum(hm, hj)
    x_ref[0, 0] = hm
    f_ref[0, 0] = jnp.broadcast_to(jnp.max(hm, axis=0, keepdims=True),
                                   (8, _CH))


def _igr_body(xp_ref, f_ref, lwt_ref, lb_ref, igwt_ref, igsb_ref,
              qkvw_ref, qkvb_ref, pw_ref, pb_ref, out_ref):
    fb = f_ref[0, :, 0, :]
    z = _dot(fb, lwt_ref[...]) + lb_ref[0, :][None, :]
    zk = z[:, :_CH]
    zq = z[:, _CH:2 * _CH]
    zv = z[:, 2 * _CH:]
    att = jax.nn.softmax(_dot(zq, zk.T), axis=0)
    wt2 = _dot(att.T, zv)
    wrow = jnp.concatenate([wt2[h:h + 1, :] for h in range(_NH)],
                           axis=1)
    acc = jnp.zeros((_N, _DIM), jnp.float32)
    for dy in range(3):
        for dx in range(3):
            x = xp_ref[0, dy:dy + _H, dx * _DIM:(dx + _W) * _DIM]
            acc = acc + _dot(x.reshape(_N, _DIM) * wrow, igwt_ref[3 * dy + dx])
    t = acc + igsb_ref[0, :][None, :]
    t = (t - igsb_ref[1, :][None, :]) \
        / jnp.sqrt(igsb_ref[2, :][None, :] + 1e-5) \
        * igsb_ref[3, :][None, :] + igsb_ref[4, :][None, :]
    vc = jnp.maximum(t, 0.0)
    qkv = _dot(vc, qkvw_ref[...]) + qkvb_ref[0, :][None, :]
    scale = float(_CH) ** (-0.5)
    outs = []
    for h in range(_NH):
        q = qkv[:, h * 192:h * 192 + _CH]
        k = qkv[:, h * 192 + _CH:h * 192 + 2 * _CH]
        v = qkv[:, h * 192 + 2 * _CH:h * 192 + 3 * _CH]
        p = jax.nn.softmax(_dot(q, k.T) * scale, axis=-1)
        outs.append(_dot(p, v))
    o = jnp.concatenate(outs, axis=1)
    out_ref[0] = _dot(o, pw_ref[...]) + pb_ref[0, :][None, :]


def _row8(v, width):
    out = jnp.zeros((8, width), jnp.float32)
    return out.at[0, :].set(v)


def _stage_v(Vx, wv, bv, bng, bnb, bnm, bnvv):
    f32 = jnp.float32
    wvt = wv.transpose(2, 3, 1, 0).reshape(9 * 1024, _DIM)
    vxp = jnp.pad(Vx.transpose(0, 2, 3, 1),
                  ((0, 0), (1, 1), (1, 1), (0, 0))).reshape(_B, 34, 34 * 1024)
    vsb = jnp.zeros((8, _DIM), f32).at[0].set(bv).at[1].set(bnm) \
        .at[2].set(bnvv).at[3].set(bng).at[4].set(bnb)

    return pl.pallas_call(
        _convv_body,
        grid=(_B,),
        in_specs=[
            pl.BlockSpec((1, 34, 34 * 1024), lambda b: (b, 0, 0)),
            pl.BlockSpec((9 * 1024, _DIM), lambda b: (0, 0)),
            pl.BlockSpec((8, _DIM), lambda b: (0, 0)),
        ],
        out_specs=pl.BlockSpec((1, _N, _DIM), lambda b: (b, 0, 0)),
        out_shape=jax.ShapeDtypeStruct((_B, _N, _DIM), f32),
    )(vxp, wvt, vsb)


def _stage_t(l, l2, l3, l4, laspp, wt_, bt):
    f32 = jnp.float32
    lt = jnp.pad(l, ((0, 0), (0, 0), (0, _TP - _T)))
    wtm = wt_[:, :, 0]
    btp = jnp.broadcast_to(bt[:, None], (_DIM, _TP))

    def _pt(a):
        return jnp.pad(a, ((0, 0), (0, 0), (0, _TP - _T)))

    lT = pl.pallas_call(
        _convt_body,
        grid=(_B,),
        in_specs=[
            pl.BlockSpec((1, 768, _TP), lambda b: (b, 0, 0)),
            pl.BlockSpec((_DIM, 768), lambda b: (0, 0)),
            pl.BlockSpec((_DIM, _TP), lambda b: (0, 0)),
            pl.BlockSpec((1, _DIM, _TP), lambda b: (b, 0, 0)),
            pl.BlockSpec((1, _DIM, _TP), lambda b: (b, 0, 0)),
            pl.BlockSpec((1, _DIM, _TP), lambda b: (b, 0, 0)),
            pl.BlockSpec((1, _DIM, _TP), lambda b: (b, 0, 0)),
        ],
        out_specs=pl.BlockSpec((1, _DIM, _TP), lambda b: (b, 0, 0)),
        out_shape=jax.ShapeDtypeStruct((_B, _DIM, _TP), f32),
    )(lt, wtm, btp, _pt(l2), _pt(l3), _pt(l4), _pt(laspp))
    return lT.transpose(0, 2, 1)


def _stage_knn(V, L, ew, eb, ebng, ebnb, ebnm, ebnv):
    f32 = jnp.float32
    V4 = V.reshape(_B, _N, _NH, _CH).transpose(0, 2, 1, 3)
    L4 = L.reshape(_B, _TP, _NH, _CH).transpose(0, 2, 1, 3)
    ewt = ew.transpose(0, 3, 4, 2, 1).reshape(_NH, 9 * 2 * _CH, _CH)
    ebp = jnp.zeros((_NH, 8, _CH), f32).at[:, 0, :].set(eb) \
        .at[:, 1, :].set(ebnm).at[:, 2, :].set(ebnv) \
        .at[:, 3, :].set(ebng).at[:, 4, :].set(ebnb)

    xs, f = pl.pallas_call(
        _knn_body,
        grid=(_B, _NH),
        in_specs=[
            pl.BlockSpec((1, 1, _N, _CH), lambda b, h: (b, h, 0, 0)),
            pl.BlockSpec((1, 1, _TP, _CH), lambda b, h: (b, h, 0, 0)),
            pl.BlockSpec((1, 9 * 2 * _CH, _CH), lambda b, h: (h, 0, 0)),
            pl.BlockSpec((1, 8, _CH), lambda b, h: (h, 0, 0)),
        ],
        out_specs=[
            pl.BlockSpec((1, 1, _N, _CH), lambda b, h: (b, h, 0, 0)),
            pl.BlockSpec((1, 1, 8, _CH), lambda b, h: (b, h, 0, 0)),
        ],
        out_shape=[
            jax.ShapeDtypeStruct((_B, _NH, _N, _CH), f32),
            jax.ShapeDtypeStruct((_B, _NH, 8, _CH), f32),
        ],
        scratch_shapes=[pltpu.VMEM((1032, (_K + 2) * 128), f32)],
    )(V4, L4, ewt, ebp)
    return xs, f


def _stage_igr(xs, f, lw, lb, igw, igb, igg, igbb, igm, igvv, qkvw, qkvb,
               pw, pb):
    f32 = jnp.float32
    x = xs.transpose(0, 2, 1, 3).reshape(_B, _H, _W, _DIM)
    xp = jnp.pad(x, ((0, 0), (1, 1), (1, 1), (0, 0))).reshape(
        _B, 34, 34 * _DIM)
    igwt = igw.transpose(2, 3, 1, 0).reshape(9, _DIM, _DIM)
    igsb = jnp.zeros((8, _DIM), f32).at[0].set(igb).at[1].set(igm) \
        .at[2].set(igvv).at[3].set(igg).at[4].set(igbb)
    qkvm = qkvw[:, :, 0, 0].T
    qkvbp = _row8(qkvb, 3 * _DIM)
    pwt = pw[:, :, 0, 0].T
    pbp = _row8(pb, _DIM)
    lwt = lw.T
    lbp = _row8(lb, 3 * _CH)

    out = pl.pallas_call(
        _igr_body,
        grid=(_B,),
        in_specs=[
            pl.BlockSpec((1, 34, 34 * _DIM), lambda b: (b, 0, 0)),
            pl.BlockSpec((1, _NH, 8, _CH), lambda b: (b, 0, 0, 0)),
            pl.BlockSpec((_CH, 3 * _CH), lambda b: (0, 0)),
            pl.BlockSpec((8, 3 * _CH), lambda b: (0, 0)),
            pl.BlockSpec((9, _DIM, _DIM), lambda b: (0, 0, 0)),
            pl.BlockSpec((8, _DIM), lambda b: (0, 0)),
            pl.BlockSpec((_DIM, 3 * _DIM), lambda b: (0, 0)),
            pl.BlockSpec((8, 3 * _DIM), lambda b: (0, 0)),
            pl.BlockSpec((_DIM, _DIM), lambda b: (0, 0)),
            pl.BlockSpec((8, _DIM), lambda b: (0, 0)),
        ],
        out_specs=pl.BlockSpec((1, _N, _DIM), lambda b: (b, 0, 0)),
        out_shape=jax.ShapeDtypeStruct((_B, _N, _DIM), f32),
    )(xp, f, lwt, lbp, igwt, igsb, qkvm, qkvbp, pwt, pbp)

    return out.transpose(0, 2, 1).reshape(_B, _DIM, _H, _W)


def kernel(Vx, l, dummy, l2, l3, l4, laspp, wv, bv, bng, bnb, bnm, bnvv, wt_,
           bt, ew, eb, ebng, ebnb, ebnm, ebnv, lw, lb, igw, igb, igg, igbb,
           igm, igvv, qkvw, qkvb, pw, pb):
    del dummy
    V = _stage_v(Vx, wv, bv, bng, bnb, bnm, bnvv)
    L = _stage_t(l, l2, l3, l4, laspp, wt_, bt)
    xs, f = _stage_knn(V, L, ew, eb, ebng, ebnb, ebnm, ebnv)
    return _stage_igr(xs, f, lw, lb, igw, igb, igg, igbb, igm, igvv,
                      qkvw, qkvb, pw, pb)
